# baseline (speedup 1.0000x reference)
import functools

import jax
import jax.numpy as jnp
from jax import lax
from jax.experimental import pallas as pl
from jax.experimental.pallas import tpu as pltpu
from jax.experimental.pallas import tpu_sc as plsc

B, L, D, E, K = 1, 2048, 768, 8, 2
F4 = 4 * D
P = L * K
TB = 256
NB = P // TB
NI = NB + E - 1
NCH = 16
CH = L // NCH

_SC = plsc.get_sparse_core_info()
NW = _SC.num_cores * _SC.num_subcores
TPW = L // NW
HALF = TPW // 2


def _router_kernel(h_ref, wr_ref, br_ref, keys_ref,
                   slot1_ref, slot2_ref, w1p_ref, w2p_ref, ends_ref):
    h = h_ref[...]
    ek = keys_ref[...]
    hn = jnp.sum(h * h, axis=1, keepdims=True)
    kn = jnp.sum(ek * ek, axis=1)[None, :]
    cross = lax.dot_general(h, ek, (((1,), (1,)), ((), ())),
                            preferred_element_type=jnp.float32)
    sq = hn + kn - 2.0 * cross
    dist = jnp.sqrt(jnp.maximum(sq, 0.0))
    logits = -dist * dist + jnp.dot(
        h, wr_ref[...], preferred_element_type=jnp.float32) + br_ref[...]
    lane = lax.broadcasted_iota(jnp.int32, (L, E), 1)
    m1 = jnp.max(logits, axis=1, keepdims=True)
    i1 = jnp.argmax(logits, axis=1)[:, None]
    masked = jnp.where(lane == i1, -jnp.inf, logits)
    m2 = jnp.max(masked, axis=1, keepdims=True)
    i2 = jnp.argmax(masked, axis=1)[:, None]
    e2 = jnp.exp(m2 - m1)
    denom = 1.0 + e2
    wa = 1.0 / denom
    wb = e2 / denom

    c1 = (lane == i1).astype(jnp.float32)
    c2 = (lane == i2).astype(jnp.float32)
    x = c1 + c2
    cs = jnp.concatenate(
        [jnp.sum(x[g * CH:(g + 1) * CH], axis=0, keepdims=True)
         for g in range(NCH)], axis=0)
    r16 = lax.broadcasted_iota(jnp.int32, (NCH, NCH), 0)
    k16 = lax.broadcasted_iota(jnp.int32, (NCH, NCH), 1)
    t16 = (k16 < r16).astype(jnp.float32)
    choff = jnp.dot(t16, cs, preferred_element_type=jnp.float32)
    cnt = jnp.sum(cs, axis=0, keepdims=True)
    ue1 = lax.broadcasted_iota(jnp.int32, (E, E), 0)
    ue2 = lax.broadcasted_iota(jnp.int32, (E, E), 1)
    cntb = jnp.broadcast_to(cnt, (E, E))
    off = jnp.sum(jnp.where(ue2 < ue1, cntb, 0.0), axis=1)[None, :]
    rch1 = lax.broadcasted_iota(jnp.int32, (CH, CH), 0)
    rch2 = lax.broadcasted_iota(jnp.int32, (CH, CH), 1)
    tch = (rch2 < rch1).astype(jnp.float32)
    s_chunks = [jnp.dot(tch, x[g * CH:(g + 1) * CH],
                        preferred_element_type=jnp.float32)
                + choff[g:g + 1, :] for g in range(NCH)]
    s = jnp.concatenate(s_chunks, axis=0) + off
    slot1 = jnp.sum(jnp.where(lane == i1, s, 0.0), axis=1, keepdims=True)
    slot2 = jnp.sum(jnp.where(lane == i2, s, 0.0), axis=1, keepdims=True)
    slot1_ref[...] = slot1.astype(jnp.int32)
    slot2_ref[...] = slot2.astype(jnp.int32)
    w1p_ref[...] = jnp.broadcast_to(wa, (L, 16))
    w2p_ref[...] = jnp.broadcast_to(wb, (L, 16))
    ends_ref[...] = (off + cnt).astype(jnp.int32)


def _run_router(h, Wr, br2, expert_keys):
    return pl.pallas_call(
        _router_kernel,
        in_specs=[
            pl.BlockSpec((L, D), lambda: (0, 0)),
            pl.BlockSpec((D, E), lambda: (0, 0)),
            pl.BlockSpec((1, E), lambda: (0, 0)),
            pl.BlockSpec((E, D), lambda: (0, 0)),
        ],
        out_specs=[
            pl.BlockSpec((L, 1), lambda: (0, 0)),
            pl.BlockSpec((L, 1), lambda: (0, 0)),
            pl.BlockSpec((L, 16), lambda: (0, 0)),
            pl.BlockSpec((L, 16), lambda: (0, 0)),
            pl.BlockSpec((1, E), lambda: (0, 0)),
        ],
        out_shape=[
            jax.ShapeDtypeStruct((L, 1), jnp.int32),
            jax.ShapeDtypeStruct((L, 1), jnp.int32),
            jax.ShapeDtypeStruct((L, 16), jnp.float32),
            jax.ShapeDtypeStruct((L, 16), jnp.float32),
            jax.ShapeDtypeStruct((1, E), jnp.int32),
        ],
    )(h, Wr, br2, expert_keys)


def _items_from_ends(ends):
    ends = ends.reshape(E)
    off = jnp.concatenate([jnp.zeros((1,), jnp.int32), ends[:-1]])
    blk = jnp.arange(NB, dtype=jnp.int32) * TB
    cands = jnp.concatenate([blk, off[1:]])
    starts = jnp.sort(cands)
    stops = jnp.concatenate([starts[1:], jnp.array([P], jnp.int32)])
    b = jnp.minimum(starts // TB, NB - 1)
    e = jnp.sum((off[1:, None] <= starts[None, :]).astype(jnp.int32), axis=0)
    lo = starts - b * TB
    hi = stops - b * TB
    return jnp.stack([b, e, lo, hi]).astype(jnp.int32)


_MESH = plsc.VectorSubcoreMesh(core_axis_name="c", subcore_axis_name="s")


@functools.partial(
    pl.kernel, mesh=_MESH,
    out_type=jax.ShapeDtypeStruct((P, D), jnp.float32),
    scratch_types=[pltpu.VMEM((TPW,), jnp.int32),
                   pltpu.VMEM((TPW,), jnp.int32),
                   pltpu.VMEM((TPW, D), jnp.float32),
                   pltpu.SemaphoreType.DMA],
)
def _sc_scatter(h_hbm, s1_hbm, s2_hbm,
                xs_hbm, idx1_v, idx2_v, rows_v, sem):
    wid = lax.axis_index("s") * _SC.num_cores + lax.axis_index("c")
    base = wid * TPW
    pltpu.sync_copy(s1_hbm.at[pl.ds(base, TPW)], idx1_v)
    pltpu.sync_copy(s2_hbm.at[pl.ds(base, TPW)], idx2_v)
    pltpu.sync_copy(h_hbm.at[pl.ds(base, TPW)], rows_v)
    c1 = pltpu.async_copy(rows_v, xs_hbm.at[idx1_v], sem)
    c2 = pltpu.async_copy(rows_v, xs_hbm.at[idx2_v], sem)
    c1.wait()
    c2.wait()


FT = 768
NF = F4 // FT


def _ffn_kernel(items_ref, xs_ref, w1_ref, b1_ref, w2_ref, b2_ref,
                ys_ref):
    j = pl.program_id(0)
    f = pl.program_id(1)
    lo = items_ref[2, j]
    hi = items_ref[3, j]
    x = xs_ref[...]
    pre = jnp.dot(x, w1_ref[0], preferred_element_type=jnp.float32) + b1_ref[0]
    hid = 0.5 * pre * (1.0 + lax.erf(pre * 0.7071067811865476))
    part = jnp.dot(hid, w2_ref[0], preferred_element_type=jnp.float32)
    part = part + jnp.where(f == 0, 1.0, 0.0) * b2_ref[0]
    riota = lax.broadcasted_iota(jnp.int32, (TB, D), 0)
    mask = jnp.logical_and(riota >= lo, riota < hi)
    prev = jnp.where(f == 0, 0.0, 1.0) * ys_ref[...]
    ys_ref[...] = jnp.where(mask, prev + part, ys_ref[...])


def _run_ffn(items, xs, W1, b1r, W2, b2r):
    grid_spec = pltpu.PrefetchScalarGridSpec(
        num_scalar_prefetch=1,
        grid=(NI, NF),
        in_specs=[
            pl.BlockSpec((TB, D), lambda j, f, it: (it[0, j], 0)),
            pl.BlockSpec((1, D, FT), lambda j, f, it: (it[1, j], 0, f)),
            pl.BlockSpec((1, 1, FT), lambda j, f, it: (it[1, j], 0, f)),
            pl.BlockSpec((1, FT, D), lambda j, f, it: (it[1, j], f, 0)),
            pl.BlockSpec((1, 1, D), lambda j, f, it: (it[1, j], 0, 0)),
        ],
        out_specs=pl.BlockSpec((TB, D), lambda j, f, it: (it[0, j], 0)),
    )
    return pl.pallas_call(
        _ffn_kernel,
        grid_spec=grid_spec,
        out_shape=jax.ShapeDtypeStruct((P, D), jnp.float32),
    )(items, xs, W1, b1r, W2, b2r)


@functools.partial(
    pl.kernel, mesh=_MESH,
    out_type=jax.ShapeDtypeStruct((L, D), jnp.float32),
    scratch_types=[pltpu.VMEM((HALF,), jnp.int32),
                   pltpu.VMEM((HALF,), jnp.int32),
                   pltpu.VMEM((HALF, 16), jnp.float32),
                   pltpu.VMEM((HALF, 16), jnp.float32),
                   pltpu.VMEM((HALF, D), jnp.float32),
                   pltpu.VMEM((HALF, D), jnp.float32),
                   pltpu.SemaphoreType.DMA],
)
def _sc_combine(ys_hbm, s1_hbm, s2_hbm, w1p_hbm, w2p_hbm, out_hbm,
                idx1_v, idx2_v, wa_v, wb_v, buf1_v, buf2_v, sem):
    wid = lax.axis_index("s") * _SC.num_cores + lax.axis_index("c")

    def half_body(half, _):
        base = wid * TPW + half * HALF
        pltpu.sync_copy(s1_hbm.at[pl.ds(base, HALF)], idx1_v)
        pltpu.sync_copy(s2_hbm.at[pl.ds(base, HALF)], idx2_v)
        pltpu.sync_copy(w1p_hbm.at[pl.ds(base, HALF)], wa_v)
        pltpu.sync_copy(w2p_hbm.at[pl.ds(base, HALF)], wb_v)
        g1 = pltpu.async_copy(ys_hbm.at[idx1_v], buf1_v, sem)
        g2 = pltpu.async_copy(ys_hbm.at[idx2_v], buf2_v, sem)
        g1.wait()
        g2.wait()

        def row_body(r, _):
            wav = wa_v[r, :]
            wbv = wb_v[r, :]
            for c in range(D // 16):
                sl = pl.ds(c * 16, 16)
                buf1_v[r, sl] = wav * buf1_v[r, sl] + wbv * buf2_v[r, sl]
            return 0

        lax.fori_loop(0, HALF, row_body, 0)
        pltpu.sync_copy(buf1_v, out_hbm.at[pl.ds(base, HALF)])
        return 0

    lax.fori_loop(0, 2, half_body, 0)


@jax.jit
def kernel(view0, W1, b1, W2, b2, Wr, br, expert_keys):
    h = view0.reshape(L, D)
    slot1, slot2, w1p, w2p, ends = _run_router(
        h, Wr, br.reshape(1, E), expert_keys)
    items = _items_from_ends(ends)
    s1 = slot1.reshape(L)
    s2 = slot2.reshape(L)
    xs = _sc_scatter(h, s1, s2)
    ys = _run_ffn(items, xs, W1, b1.reshape(E, 1, F4), W2,
                  b2.reshape(E, 1, D))
    out = _sc_combine(ys, s1, s2, w1p, w2p)
    return out.reshape(B, L, D)


# speedup vs baseline: 1.1680x; 1.1680x over previous
import functools

import jax
import jax.numpy as jnp
from jax import lax
from jax.experimental import pallas as pl
from jax.experimental.pallas import tpu as pltpu
from jax.experimental.pallas import tpu_sc as plsc

B, L, D, E, K = 1, 2048, 768, 8, 2
F4 = 4 * D
P = L * K
TB = 256
NB = P // TB
NI = NB + E - 1
NCH = 16
CH = L // NCH

_SC = plsc.get_sparse_core_info()
NW = _SC.num_cores * _SC.num_subcores
TPW = L // NW
HALF = TPW // 2


def _router_kernel(h_ref, wr_ref, br_ref, keys_ref,
                   slot1_ref, slot2_ref, w1p_ref, w2p_ref, ends_ref):
    h = h_ref[...]
    ek = keys_ref[...]
    hn = jnp.sum(h * h, axis=1, keepdims=True)
    kn = jnp.sum(ek * ek, axis=1)[None, :]
    cross = lax.dot_general(h, ek, (((1,), (1,)), ((), ())),
                            preferred_element_type=jnp.float32)
    sq = hn + kn - 2.0 * cross
    dist = jnp.sqrt(jnp.maximum(sq, 0.0))
    logits = -dist * dist + jnp.dot(
        h, wr_ref[...], preferred_element_type=jnp.float32) + br_ref[...]
    lane = lax.broadcasted_iota(jnp.int32, (L, E), 1)
    m1 = jnp.max(logits, axis=1, keepdims=True)
    i1 = jnp.argmax(logits, axis=1)[:, None]
    masked = jnp.where(lane == i1, -jnp.inf, logits)
    m2 = jnp.max(masked, axis=1, keepdims=True)
    i2 = jnp.argmax(masked, axis=1)[:, None]
    e2 = jnp.exp(m2 - m1)
    denom = 1.0 + e2
    wa = 1.0 / denom
    wb = e2 / denom

    c1 = (lane == i1).astype(jnp.float32)
    c2 = (lane == i2).astype(jnp.float32)
    x = c1 + c2
    cs = jnp.concatenate(
        [jnp.sum(x[g * CH:(g + 1) * CH], axis=0, keepdims=True)
         for g in range(NCH)], axis=0)
    r16 = lax.broadcasted_iota(jnp.int32, (NCH, NCH), 0)
    k16 = lax.broadcasted_iota(jnp.int32, (NCH, NCH), 1)
    t16 = (k16 < r16).astype(jnp.float32)
    choff = jnp.dot(t16, cs, preferred_element_type=jnp.float32)
    cnt = jnp.sum(cs, axis=0, keepdims=True)
    ue1 = lax.broadcasted_iota(jnp.int32, (E, E), 0)
    ue2 = lax.broadcasted_iota(jnp.int32, (E, E), 1)
    cntb = jnp.broadcast_to(cnt, (E, E))
    off = jnp.sum(jnp.where(ue2 < ue1, cntb, 0.0), axis=1)[None, :]
    rch1 = lax.broadcasted_iota(jnp.int32, (CH, CH), 0)
    rch2 = lax.broadcasted_iota(jnp.int32, (CH, CH), 1)
    tch = (rch2 < rch1).astype(jnp.float32)
    s_chunks = [jnp.dot(tch, x[g * CH:(g + 1) * CH],
                        preferred_element_type=jnp.float32)
                + choff[g:g + 1, :] for g in range(NCH)]
    s = jnp.concatenate(s_chunks, axis=0) + off
    slot1 = jnp.sum(jnp.where(lane == i1, s, 0.0), axis=1, keepdims=True)
    slot2 = jnp.sum(jnp.where(lane == i2, s, 0.0), axis=1, keepdims=True)
    slot1_ref[...] = slot1.astype(jnp.int32)
    slot2_ref[...] = slot2.astype(jnp.int32)
    w1p_ref[...] = jnp.broadcast_to(wa, (L, 16))
    w2p_ref[...] = jnp.broadcast_to(wb, (L, 16))
    ends_ref[...] = (off + cnt).astype(jnp.int32)


def _run_router(h, Wr, br2, expert_keys):
    return pl.pallas_call(
        _router_kernel,
        in_specs=[
            pl.BlockSpec((L, D), lambda: (0, 0)),
            pl.BlockSpec((D, E), lambda: (0, 0)),
            pl.BlockSpec((1, E), lambda: (0, 0)),
            pl.BlockSpec((E, D), lambda: (0, 0)),
        ],
        out_specs=[
            pl.BlockSpec((L, 1), lambda: (0, 0)),
            pl.BlockSpec((L, 1), lambda: (0, 0)),
            pl.BlockSpec((L, 16), lambda: (0, 0)),
            pl.BlockSpec((L, 16), lambda: (0, 0)),
            pl.BlockSpec((1, E), lambda: (0, 0)),
        ],
        out_shape=[
            jax.ShapeDtypeStruct((L, 1), jnp.int32),
            jax.ShapeDtypeStruct((L, 1), jnp.int32),
            jax.ShapeDtypeStruct((L, 16), jnp.float32),
            jax.ShapeDtypeStruct((L, 16), jnp.float32),
            jax.ShapeDtypeStruct((1, E), jnp.int32),
        ],
    )(h, Wr, br2, expert_keys)


def _items_from_ends(ends):
    ends = ends.reshape(E)
    off = jnp.concatenate([jnp.zeros((1,), jnp.int32), ends[:-1]])
    blk = jnp.arange(NB, dtype=jnp.int32) * TB
    cands = jnp.concatenate([blk, off[1:]])
    starts = jnp.sort(cands)
    stops = jnp.concatenate([starts[1:], jnp.array([P], jnp.int32)])
    b = jnp.minimum(starts // TB, NB - 1)
    e = jnp.sum((off[1:, None] <= starts[None, :]).astype(jnp.int32), axis=0)
    lo = starts - b * TB
    hi = stops - b * TB
    return jnp.stack([b, e, lo, hi]).astype(jnp.int32)


_MESH = plsc.VectorSubcoreMesh(core_axis_name="c", subcore_axis_name="s")


@functools.partial(
    pl.kernel, mesh=_MESH,
    out_type=jax.ShapeDtypeStruct((P, D), jnp.float32),
    scratch_types=[pltpu.VMEM((TPW,), jnp.int32),
                   pltpu.VMEM((TPW,), jnp.int32),
                   pltpu.VMEM((TPW, D), jnp.float32),
                   pltpu.SemaphoreType.DMA],
)
def _sc_scatter(h_hbm, s1_hbm, s2_hbm,
                xs_hbm, idx1_v, idx2_v, rows_v, sem):
    wid = lax.axis_index("s") * _SC.num_cores + lax.axis_index("c")
    base = wid * TPW
    pltpu.sync_copy(s1_hbm.at[pl.ds(base, TPW)], idx1_v)
    pltpu.sync_copy(s2_hbm.at[pl.ds(base, TPW)], idx2_v)
    pltpu.sync_copy(h_hbm.at[pl.ds(base, TPW)], rows_v)
    c1 = pltpu.async_copy(rows_v, xs_hbm.at[idx1_v], sem)
    c2 = pltpu.async_copy(rows_v, xs_hbm.at[idx2_v], sem)
    c1.wait()
    c2.wait()


def _ffn_kernel(items_ref, xs_ref, w1_ref, b1_ref, w2_ref, b2_ref,
                ys_ref):
    j = pl.program_id(0)
    lo = items_ref[2, j]
    hi = items_ref[3, j]
    x = xs_ref[...]
    pre = jnp.dot(x, w1_ref[0], preferred_element_type=jnp.float32) + b1_ref[0]
    hid = 0.5 * pre * (1.0 + lax.erf(pre * 0.7071067811865476))
    y = jnp.dot(hid, w2_ref[0], preferred_element_type=jnp.float32) + b2_ref[0]
    riota = lax.broadcasted_iota(jnp.int32, (TB, D), 0)
    mask = jnp.logical_and(riota >= lo, riota < hi)
    ys_ref[...] = jnp.where(mask, y, ys_ref[...])


def _run_ffn(items, xs, W1, b1r, W2, b2r):
    grid_spec = pltpu.PrefetchScalarGridSpec(
        num_scalar_prefetch=1,
        grid=(NI,),
        in_specs=[
            pl.BlockSpec((TB, D), lambda j, it: (it[0, j], 0)),
            pl.BlockSpec((1, D, F4), lambda j, it: (it[1, j], 0, 0)),
            pl.BlockSpec((1, 1, F4), lambda j, it: (it[1, j], 0, 0)),
            pl.BlockSpec((1, F4, D), lambda j, it: (it[1, j], 0, 0)),
            pl.BlockSpec((1, 1, D), lambda j, it: (it[1, j], 0, 0)),
        ],
        out_specs=pl.BlockSpec((TB, D), lambda j, it: (it[0, j], 0)),
    )
    return pl.pallas_call(
        _ffn_kernel,
        grid_spec=grid_spec,
        out_shape=jax.ShapeDtypeStruct((P, D), jnp.float32),
    )(items, xs, W1, b1r, W2, b2r)


@functools.partial(
    pl.kernel, mesh=_MESH,
    out_type=jax.ShapeDtypeStruct((L, D), jnp.float32),
    scratch_types=[pltpu.VMEM((HALF,), jnp.int32),
                   pltpu.VMEM((HALF,), jnp.int32),
                   pltpu.VMEM((HALF, 16), jnp.float32),
                   pltpu.VMEM((HALF, 16), jnp.float32),
                   pltpu.VMEM((HALF, D), jnp.float32),
                   pltpu.VMEM((HALF, D), jnp.float32),
                   pltpu.SemaphoreType.DMA],
)
def _sc_combine(ys_hbm, s1_hbm, s2_hbm, w1p_hbm, w2p_hbm, out_hbm,
                idx1_v, idx2_v, wa_v, wb_v, buf1_v, buf2_v, sem):
    wid = lax.axis_index("s") * _SC.num_cores + lax.axis_index("c")

    def half_body(half, _):
        base = wid * TPW + half * HALF
        pltpu.sync_copy(s1_hbm.at[pl.ds(base, HALF)], idx1_v)
        pltpu.sync_copy(s2_hbm.at[pl.ds(base, HALF)], idx2_v)
        pltpu.sync_copy(w1p_hbm.at[pl.ds(base, HALF)], wa_v)
        pltpu.sync_copy(w2p_hbm.at[pl.ds(base, HALF)], wb_v)
        g1 = pltpu.async_copy(ys_hbm.at[idx1_v], buf1_v, sem)
        g2 = pltpu.async_copy(ys_hbm.at[idx2_v], buf2_v, sem)
        g1.wait()
        g2.wait()

        def row_body(r, _):
            wav = wa_v[r, :]
            wbv = wb_v[r, :]
            for c in range(D // 16):
                sl = pl.ds(c * 16, 16)
                buf1_v[r, sl] = wav * buf1_v[r, sl] + wbv * buf2_v[r, sl]
            return 0

        lax.fori_loop(0, HALF, row_body, 0)
        pltpu.sync_copy(buf1_v, out_hbm.at[pl.ds(base, HALF)])
        return 0

    lax.fori_loop(0, 2, half_body, 0)


@jax.jit
def kernel(view0, W1, b1, W2, b2, Wr, br, expert_keys):
    h = view0.reshape(L, D)
    slot1, slot2, w1p, w2p, ends = _run_router(
        h, Wr, br.reshape(1, E), expert_keys)
    items = _items_from_ends(ends)
    s1 = slot1.reshape(L)
    s2 = slot2.reshape(L)
    xs = jnp.zeros((P, D), jnp.float32).at[s1].set(h).at[s2].set(h)
    ys = _run_ffn(items, xs, W1, b1.reshape(E, 1, F4), W2,
                  b2.reshape(E, 1, D))
    out = ys[s1] * w1p[:, 0:1] + ys[s2] * w2p[:, 0:1]
    return out.reshape(B, L, D)


# speedup vs baseline: 1.4682x; 1.2570x over previous
import functools

import jax
import jax.numpy as jnp
from jax import lax
from jax.experimental import pallas as pl
from jax.experimental.pallas import tpu as pltpu
from jax.experimental.pallas import tpu_sc as plsc

B, L, D, E, K = 1, 2048, 768, 8, 2
F4 = 4 * D
P = L * K
TB = 512
NB = P // TB
NI = NB + E - 1
NCH = 16
CH = L // NCH

_SC = plsc.get_sparse_core_info()
NW = _SC.num_cores * _SC.num_subcores
TPW = L // NW
HALF = TPW // 2


def _router_kernel(h_ref, wr_ref, br_ref, keys_ref,
                   slot1_ref, slot2_ref, w1p_ref, w2p_ref, ends_ref):
    h = h_ref[...]
    ek = keys_ref[...]
    hn = jnp.sum(h * h, axis=1, keepdims=True)
    kn = jnp.sum(ek * ek, axis=1)[None, :]
    cross = lax.dot_general(h, ek, (((1,), (1,)), ((), ())),
                            preferred_element_type=jnp.float32)
    sq = hn + kn - 2.0 * cross
    dist = jnp.sqrt(jnp.maximum(sq, 0.0))
    logits = -dist * dist + jnp.dot(
        h, wr_ref[...], preferred_element_type=jnp.float32) + br_ref[...]
    lane = lax.broadcasted_iota(jnp.int32, (L, E), 1)
    m1 = jnp.max(logits, axis=1, keepdims=True)
    i1 = jnp.argmax(logits, axis=1)[:, None]
    masked = jnp.where(lane == i1, -jnp.inf, logits)
    m2 = jnp.max(masked, axis=1, keepdims=True)
    i2 = jnp.argmax(masked, axis=1)[:, None]
    e2 = jnp.exp(m2 - m1)
    denom = 1.0 + e2
    wa = 1.0 / denom
    wb = e2 / denom

    c1 = (lane == i1).astype(jnp.float32)
    c2 = (lane == i2).astype(jnp.float32)
    x = c1 + c2
    cs = jnp.concatenate(
        [jnp.sum(x[g * CH:(g + 1) * CH], axis=0, keepdims=True)
         for g in range(NCH)], axis=0)
    r16 = lax.broadcasted_iota(jnp.int32, (NCH, NCH), 0)
    k16 = lax.broadcasted_iota(jnp.int32, (NCH, NCH), 1)
    t16 = (k16 < r16).astype(jnp.float32)
    choff = jnp.dot(t16, cs, preferred_element_type=jnp.float32)
    cnt = jnp.sum(cs, axis=0, keepdims=True)
    ue1 = lax.broadcasted_iota(jnp.int32, (E, E), 0)
    ue2 = lax.broadcasted_iota(jnp.int32, (E, E), 1)
    cntb = jnp.broadcast_to(cnt, (E, E))
    off = jnp.sum(jnp.where(ue2 < ue1, cntb, 0.0), axis=1)[None, :]
    rch1 = lax.broadcasted_iota(jnp.int32, (CH, CH), 0)
    rch2 = lax.broadcasted_iota(jnp.int32, (CH, CH), 1)
    tch = (rch2 < rch1).astype(jnp.float32)
    s_chunks = [jnp.dot(tch, x[g * CH:(g + 1) * CH],
                        preferred_element_type=jnp.float32)
                + choff[g:g + 1, :] for g in range(NCH)]
    s = jnp.concatenate(s_chunks, axis=0) + off
    slot1 = jnp.sum(jnp.where(lane == i1, s, 0.0), axis=1, keepdims=True)
    slot2 = jnp.sum(jnp.where(lane == i2, s, 0.0), axis=1, keepdims=True)
    slot1_ref[...] = slot1.astype(jnp.int32)
    slot2_ref[...] = slot2.astype(jnp.int32)
    w1p_ref[...] = jnp.broadcast_to(wa, (L, 16))
    w2p_ref[...] = jnp.broadcast_to(wb, (L, 16))
    ends_ref[...] = (off + cnt).astype(jnp.int32)


def _run_router(h, Wr, br2, expert_keys):
    return pl.pallas_call(
        _router_kernel,
        in_specs=[
            pl.BlockSpec((L, D), lambda: (0, 0)),
            pl.BlockSpec((D, E), lambda: (0, 0)),
            pl.BlockSpec((1, E), lambda: (0, 0)),
            pl.BlockSpec((E, D), lambda: (0, 0)),
        ],
        out_specs=[
            pl.BlockSpec((L, 1), lambda: (0, 0)),
            pl.BlockSpec((L, 1), lambda: (0, 0)),
            pl.BlockSpec((L, 16), lambda: (0, 0)),
            pl.BlockSpec((L, 16), lambda: (0, 0)),
            pl.BlockSpec((1, E), lambda: (0, 0)),
        ],
        out_shape=[
            jax.ShapeDtypeStruct((L, 1), jnp.int32),
            jax.ShapeDtypeStruct((L, 1), jnp.int32),
            jax.ShapeDtypeStruct((L, 16), jnp.float32),
            jax.ShapeDtypeStruct((L, 16), jnp.float32),
            jax.ShapeDtypeStruct((1, E), jnp.int32),
        ],
    )(h, Wr, br2, expert_keys)


def _items_from_ends(ends):
    ends = ends.reshape(E)
    off = jnp.concatenate([jnp.zeros((1,), jnp.int32), ends[:-1]])
    blk = jnp.arange(NB, dtype=jnp.int32) * TB
    cands = jnp.concatenate([blk, off[1:]])
    starts = jnp.sort(cands)
    stops = jnp.concatenate([starts[1:], jnp.array([P], jnp.int32)])
    b = jnp.minimum(starts // TB, NB - 1)
    e = jnp.sum((off[1:, None] <= starts[None, :]).astype(jnp.int32), axis=0)
    lo = starts - b * TB
    hi = stops - b * TB
    return jnp.stack([b, e, lo, hi]).astype(jnp.int32)


_MESH = plsc.VectorSubcoreMesh(core_axis_name="c", subcore_axis_name="s")


@functools.partial(
    pl.kernel, mesh=_MESH,
    out_type=jax.ShapeDtypeStruct((P, D), jnp.float32),
    scratch_types=[pltpu.VMEM((TPW,), jnp.int32),
                   pltpu.VMEM((TPW,), jnp.int32),
                   pltpu.VMEM((TPW, D), jnp.float32),
                   pltpu.SemaphoreType.DMA],
)
def _sc_scatter(h_hbm, s1_hbm, s2_hbm,
                xs_hbm, idx1_v, idx2_v, rows_v, sem):
    wid = lax.axis_index("s") * _SC.num_cores + lax.axis_index("c")
    base = wid * TPW
    pltpu.sync_copy(s1_hbm.at[pl.ds(base, TPW)], idx1_v)
    pltpu.sync_copy(s2_hbm.at[pl.ds(base, TPW)], idx2_v)
    pltpu.sync_copy(h_hbm.at[pl.ds(base, TPW)], rows_v)
    c1 = pltpu.async_copy(rows_v, xs_hbm.at[idx1_v], sem)
    c2 = pltpu.async_copy(rows_v, xs_hbm.at[idx2_v], sem)
    c1.wait()
    c2.wait()


def _ffn_kernel(items_ref, xs_ref, w1_ref, b1_ref, w2_ref, b2_ref,
                ys_ref):
    j = pl.program_id(0)
    lo = items_ref[2, j]
    hi = items_ref[3, j]
    x = xs_ref[...]
    pre = jnp.dot(x, w1_ref[0], preferred_element_type=jnp.float32) + b1_ref[0]
    hid = 0.5 * pre * (1.0 + lax.erf(pre * 0.7071067811865476))
    y = jnp.dot(hid, w2_ref[0], preferred_element_type=jnp.float32) + b2_ref[0]
    riota = lax.broadcasted_iota(jnp.int32, (TB, D), 0)
    mask = jnp.logical_and(riota >= lo, riota < hi)
    ys_ref[...] = jnp.where(mask, y, ys_ref[...])


def _run_ffn(items, xs, W1, b1r, W2, b2r):
    grid_spec = pltpu.PrefetchScalarGridSpec(
        num_scalar_prefetch=1,
        grid=(NI,),
        in_specs=[
            pl.BlockSpec((TB, D), lambda j, it: (it[0, j], 0)),
            pl.BlockSpec((1, D, F4), lambda j, it: (it[1, j], 0, 0)),
            pl.BlockSpec((1, 1, F4), lambda j, it: (it[1, j], 0, 0)),
            pl.BlockSpec((1, F4, D), lambda j, it: (it[1, j], 0, 0)),
            pl.BlockSpec((1, 1, D), lambda j, it: (it[1, j], 0, 0)),
        ],
        out_specs=pl.BlockSpec((TB, D), lambda j, it: (it[0, j], 0)),
    )
    return pl.pallas_call(
        _ffn_kernel,
        grid_spec=grid_spec,
        out_shape=jax.ShapeDtypeStruct((P, D), jnp.float32),
    )(items, xs, W1, b1r, W2, b2r)


@functools.partial(
    pl.kernel, mesh=_MESH,
    out_type=jax.ShapeDtypeStruct((L, D), jnp.float32),
    scratch_types=[pltpu.VMEM((HALF,), jnp.int32),
                   pltpu.VMEM((HALF,), jnp.int32),
                   pltpu.VMEM((HALF, 16), jnp.float32),
                   pltpu.VMEM((HALF, 16), jnp.float32),
                   pltpu.VMEM((HALF, D), jnp.float32),
                   pltpu.VMEM((HALF, D), jnp.float32),
                   pltpu.SemaphoreType.DMA],
)
def _sc_combine(ys_hbm, s1_hbm, s2_hbm, w1p_hbm, w2p_hbm, out_hbm,
                idx1_v, idx2_v, wa_v, wb_v, buf1_v, buf2_v, sem):
    wid = lax.axis_index("s") * _SC.num_cores + lax.axis_index("c")

    def half_body(half, _):
        base = wid * TPW + half * HALF
        pltpu.sync_copy(s1_hbm.at[pl.ds(base, HALF)], idx1_v)
        pltpu.sync_copy(s2_hbm.at[pl.ds(base, HALF)], idx2_v)
        pltpu.sync_copy(w1p_hbm.at[pl.ds(base, HALF)], wa_v)
        pltpu.sync_copy(w2p_hbm.at[pl.ds(base, HALF)], wb_v)
        g1 = pltpu.async_copy(ys_hbm.at[idx1_v], buf1_v, sem)
        g2 = pltpu.async_copy(ys_hbm.at[idx2_v], buf2_v, sem)
        g1.wait()
        g2.wait()

        def row_body(r, _):
            wav = wa_v[r, :]
            wbv = wb_v[r, :]
            for c in range(D // 16):
                sl = pl.ds(c * 16, 16)
                buf1_v[r, sl] = wav * buf1_v[r, sl] + wbv * buf2_v[r, sl]
            return 0

        lax.fori_loop(0, HALF, row_body, 0)
        pltpu.sync_copy(buf1_v, out_hbm.at[pl.ds(base, HALF)])
        return 0

    lax.fori_loop(0, 2, half_body, 0)


@jax.jit
def kernel(view0, W1, b1, W2, b2, Wr, br, expert_keys):
    h = view0.reshape(L, D)
    slot1, slot2, w1p, w2p, ends = _run_router(
        h, Wr, br.reshape(1, E), expert_keys)
    items = _items_from_ends(ends)
    s1 = slot1.reshape(L)
    s2 = slot2.reshape(L)
    xs = _sc_scatter(h, s1, s2)
    ys = _run_ffn(items, xs, W1, b1.reshape(E, 1, F4), W2,
                  b2.reshape(E, 1, D))
    out = _sc_combine(ys, s1, s2, w1p, w2p)
    return out.reshape(B, L, D)
